# baseline (speedup 1.0000x reference)
import jax
import jax.numpy as jnp
from jax.experimental import pallas as pl
from jax.experimental.pallas import tpu as pltpu

TOKENS = 32
INPUT_DIM = 1024
DICT_SIZE = 32768
BLOCK_D = 2048
NCORE = 2
STEPS = DICT_SIZE // BLOCK_D // NCORE


def _fused_body(x_ref, we_ref, be_ref, wd_ref, bd_ref, xhat_ref, z_ref):
    i = pl.program_id(1)
    pre = jax.lax.dot_general(
        x_ref[...], we_ref[...],
        dimension_numbers=(((1,), (1,)), ((), ())),
        preferred_element_type=jnp.float32,
    )
    z = jnp.maximum(pre + be_ref[...], 0.0)
    z_ref[...] = z
    part = jax.lax.dot_general(
        z, wd_ref[...],
        dimension_numbers=(((1,), (1,)), ((), ())),
        preferred_element_type=jnp.float32,
    )

    @pl.when(i == 0)
    def _init():
        xhat_ref[...] = part[None]

    @pl.when(i > 0)
    def _acc():
        xhat_ref[...] += part[None]


def kernel(x, W_enc, b_enc, W_dec, b_dec):
    b_enc2 = b_enc.reshape(1, DICT_SIZE)
    b_dec2 = b_dec.reshape(1, INPUT_DIM)
    grid = (NCORE, STEPS)
    xhat_part, z = pl.pallas_call(
        _fused_body,
        grid=grid,
        in_specs=[
            pl.BlockSpec((TOKENS, INPUT_DIM), lambda c, i: (0, 0)),
            pl.BlockSpec((BLOCK_D, INPUT_DIM), lambda c, i: (c * STEPS + i, 0)),
            pl.BlockSpec((1, BLOCK_D), lambda c, i: (0, c * STEPS + i)),
            pl.BlockSpec((INPUT_DIM, BLOCK_D), lambda c, i: (0, c * STEPS + i)),
            pl.BlockSpec((1, INPUT_DIM), lambda c, i: (0, 0)),
        ],
        out_specs=[
            pl.BlockSpec((1, TOKENS, INPUT_DIM), lambda c, i: (c, 0, 0)),
            pl.BlockSpec((TOKENS, BLOCK_D), lambda c, i: (0, c * STEPS + i)),
        ],
        out_shape=[
            jax.ShapeDtypeStruct((NCORE, TOKENS, INPUT_DIM), jnp.float32),
            jax.ShapeDtypeStruct((TOKENS, DICT_SIZE), jnp.float32),
        ],
        compiler_params=pltpu.CompilerParams(
            dimension_semantics=("parallel", "arbitrary"),
        ),
    )(x, W_enc, b_enc2, W_dec, b_dec2)
    x_hat = xhat_part[0] + xhat_part[1] + b_dec2
    return (x_hat, z)


# speedup vs baseline: 1.0229x; 1.0229x over previous
import jax
import jax.numpy as jnp
from jax.experimental import pallas as pl
from jax.experimental.pallas import tpu as pltpu

TOKENS = 32
INPUT_DIM = 1024
DICT_SIZE = 32768
BLOCK_D = 1024


def _fused_body(x_ref, we_ref, be_ref, wd_ref, bd_ref, xhat_ref, z_ref):
    i = pl.program_id(0)
    pre = jax.lax.dot_general(
        x_ref[...], we_ref[...],
        dimension_numbers=(((1,), (1,)), ((), ())),
        preferred_element_type=jnp.float32,
    )
    z = jnp.maximum(pre + be_ref[...], 0.0)
    z_ref[...] = z
    part = jax.lax.dot_general(
        z, wd_ref[...],
        dimension_numbers=(((1,), (1,)), ((), ())),
        preferred_element_type=jnp.float32,
    )

    @pl.when(i == 0)
    def _init():
        xhat_ref[...] = part + bd_ref[...]

    @pl.when(i > 0)
    def _acc():
        xhat_ref[...] += part


def kernel(x, W_enc, b_enc, W_dec, b_dec):
    b_enc2 = b_enc.reshape(1, DICT_SIZE)
    b_dec2 = b_dec.reshape(1, INPUT_DIM)
    grid = (DICT_SIZE // BLOCK_D,)
    x_hat, z = pl.pallas_call(
        _fused_body,
        grid=grid,
        in_specs=[
            pl.BlockSpec((TOKENS, INPUT_DIM), lambda i: (0, 0)),
            pl.BlockSpec((BLOCK_D, INPUT_DIM), lambda i: (i, 0)),
            pl.BlockSpec((1, BLOCK_D), lambda i: (0, i)),
            pl.BlockSpec((INPUT_DIM, BLOCK_D), lambda i: (0, i)),
            pl.BlockSpec((1, INPUT_DIM), lambda i: (0, 0)),
        ],
        out_specs=[
            pl.BlockSpec((TOKENS, INPUT_DIM), lambda i: (0, 0)),
            pl.BlockSpec((TOKENS, BLOCK_D), lambda i: (0, i)),
        ],
        out_shape=[
            jax.ShapeDtypeStruct((TOKENS, INPUT_DIM), jnp.float32),
            jax.ShapeDtypeStruct((TOKENS, DICT_SIZE), jnp.float32),
        ],
        compiler_params=pltpu.CompilerParams(
            dimension_semantics=("arbitrary",),
        ),
    )(x, W_enc, b_enc2, W_dec, b_dec2)
    return (x_hat, z)


# speedup vs baseline: 1.0248x; 1.0018x over previous
import jax
import jax.numpy as jnp
from jax.experimental import pallas as pl
from jax.experimental.pallas import tpu as pltpu

TOKENS = 32
INPUT_DIM = 1024
DICT_SIZE = 32768
BLOCK_D = 1024


def _fused_body(x_ref, wea_ref, web_ref, be_a, be_b, wda_ref, wdb_ref, bd_ref,
                xhat_ref, z_ref):
    i = pl.program_id(0)
    dn = (((1,), (1,)), ((), ()))
    za = jnp.maximum(
        jax.lax.dot_general(x_ref[...], wea_ref[...], dn,
                            preferred_element_type=jnp.float32) + be_a[...], 0.0)
    zb = jnp.maximum(
        jax.lax.dot_general(x_ref[...], web_ref[...], dn,
                            preferred_element_type=jnp.float32) + be_b[...], 0.0)
    z_ref[:, :BLOCK_D] = za
    z_ref[:, BLOCK_D:] = zb
    part = jax.lax.dot_general(za, wda_ref[...], dn,
                               preferred_element_type=jnp.float32)
    part += jax.lax.dot_general(zb, wdb_ref[...], dn,
                                preferred_element_type=jnp.float32)

    @pl.when(i == 0)
    def _init():
        xhat_ref[...] = part + bd_ref[...]

    @pl.when(i > 0)
    def _acc():
        xhat_ref[...] += part


def kernel(x, W_enc, b_enc, W_dec, b_dec):
    b_enc2 = b_enc.reshape(1, DICT_SIZE)
    b_dec2 = b_dec.reshape(1, INPUT_DIM)
    grid = (DICT_SIZE // BLOCK_D // 2,)
    x_hat, z = pl.pallas_call(
        _fused_body,
        grid=grid,
        in_specs=[
            pl.BlockSpec((TOKENS, INPUT_DIM), lambda i: (0, 0)),
            pl.BlockSpec((BLOCK_D, INPUT_DIM), lambda i: (2 * i, 0)),
            pl.BlockSpec((BLOCK_D, INPUT_DIM), lambda i: (2 * i + 1, 0)),
            pl.BlockSpec((1, BLOCK_D), lambda i: (0, 2 * i)),
            pl.BlockSpec((1, BLOCK_D), lambda i: (0, 2 * i + 1)),
            pl.BlockSpec((INPUT_DIM, BLOCK_D), lambda i: (0, 2 * i)),
            pl.BlockSpec((INPUT_DIM, BLOCK_D), lambda i: (0, 2 * i + 1)),
            pl.BlockSpec((1, INPUT_DIM), lambda i: (0, 0)),
        ],
        out_specs=[
            pl.BlockSpec((TOKENS, INPUT_DIM), lambda i: (0, 0)),
            pl.BlockSpec((TOKENS, 2 * BLOCK_D), lambda i: (0, i)),
        ],
        out_shape=[
            jax.ShapeDtypeStruct((TOKENS, INPUT_DIM), jnp.float32),
            jax.ShapeDtypeStruct((TOKENS, DICT_SIZE), jnp.float32),
        ],
        compiler_params=pltpu.CompilerParams(
            dimension_semantics=("arbitrary",),
        ),
    )(x, W_enc, W_enc, b_enc2, b_enc2, W_dec, W_dec, b_dec2)
    return (x_hat, z)
